# baseline (speedup 1.0000x reference)
import functools

import jax
import jax.numpy as jnp
from jax.experimental import pallas as pl
from jax.experimental.pallas import tpu as pltpu


def _gcn_kernel(x_ref, a_ref, w1_ref, b1_ref, w2_ref, b2_ref, out_ref,
                g_ref, acache_ref, *, bm, nm, ck, f):
    i = pl.program_id(0)

    @pl.when(i < nm)
    def _():
        m = i
        t = jnp.dot(a_ref[...].astype(jnp.bfloat16),
                    x_ref[...].astype(jnp.bfloat16),
                    preferred_element_type=jnp.float32)
        s = jnp.dot(t, w1_ref[...], preferred_element_type=jnp.float32)
        s = jnp.maximum(s + b1_ref[...], 0.0)
        g_ref[pl.ds(m * bm, bm), :] = jnp.dot(
            s, w2_ref[...],
            preferred_element_type=jnp.float32).astype(jnp.bfloat16)

        @pl.when(jnp.logical_and(m >= nm - 1 - ck, m <= nm - 2))
        def _():
            base = (m - (nm - 1 - ck)) * bm
            nn = nm * bm
            for c in range(0, nn, 2000):
                hi = min(c + 2000, nn)
                acache_ref[pl.ds(base, bm), c:hi] = (
                    a_ref[:, c:hi].astype(jnp.bfloat16))

    @pl.when(i >= nm)
    def _():
        j = i - nm
        gb = g_ref[...]

        @pl.when(jnp.logical_or(j == 0, j > ck))
        def _():
            s = jnp.dot(a_ref[...].astype(jnp.bfloat16), gb,
                        preferred_element_type=jnp.float32)
            out_ref[...] = jnp.maximum(s + b2_ref[...], 0.0)

        @pl.when(jnp.logical_and(j >= 1, j <= ck))
        def _():
            ab = acache_ref[pl.ds((ck - j) * bm, bm), :]
            s = jnp.dot(ab, gb, preferred_element_type=jnp.float32)
            out_ref[...] = jnp.maximum(s + b2_ref[...], 0.0)


def _phase1_m(j, nm, ck):
    return jnp.where(j <= ck, nm - 1 - j, j - ck - 1)


def kernel(x, a, W1, b1, W2, b2):
    n, f_in = x.shape
    f = W1.shape[1]
    bm = 400
    nm = n // bm
    ck = min(2, nm - 2)

    def a_map(i, nm=nm, ck=ck):
        j = i - nm
        return (jnp.where(i < nm, i,
                          jnp.where(j <= ck, nm - 1, j - ck - 1)), 0)

    def out_map(i, nm=nm, ck=ck):
        return (jnp.where(i < nm, nm - 1, _phase1_m(i - nm, nm, ck)), 0)

    return pl.pallas_call(
        functools.partial(_gcn_kernel, bm=bm, nm=nm, ck=ck, f=f),
        grid=(2 * nm,),
        in_specs=[
            pl.BlockSpec((n, f_in), lambda i: (0, 0),
                         pipeline_mode=pl.Buffered(buffer_count=1)),
            pl.BlockSpec((bm, n), a_map,
                         pipeline_mode=pl.Buffered(buffer_count=2)),
            pl.BlockSpec((f_in, f), lambda i: (0, 0),
                         pipeline_mode=pl.Buffered(buffer_count=1)),
            pl.BlockSpec((1, f), lambda i: (0, 0),
                         pipeline_mode=pl.Buffered(buffer_count=1)),
            pl.BlockSpec((f, f), lambda i: (0, 0),
                         pipeline_mode=pl.Buffered(buffer_count=1)),
            pl.BlockSpec((1, f), lambda i: (0, 0),
                         pipeline_mode=pl.Buffered(buffer_count=1)),
        ],
        out_specs=pl.BlockSpec((bm, f), out_map,
                               pipeline_mode=pl.Buffered(buffer_count=1)),
        out_shape=jax.ShapeDtypeStruct((n, f), jnp.float32),
        scratch_shapes=[
            pltpu.VMEM((n, f), jnp.bfloat16),
            pltpu.VMEM((ck * bm, n), jnp.bfloat16),
        ],
        compiler_params=pltpu.CompilerParams(
            dimension_semantics=("arbitrary",),
            vmem_limit_bytes=64 * 1024 * 1024,
            internal_scratch_in_bytes=256 * 1024,
        ),
    )(x, a, W1, b1.reshape(1, f), W2, b2.reshape(1, f))


# speedup vs baseline: 1.0090x; 1.0090x over previous
import functools

import jax
import jax.numpy as jnp
from jax.experimental import pallas as pl
from jax.experimental.pallas import tpu as pltpu


def _gcn_kernel(x_ref, a_ref, w1_ref, b1_ref, w2_ref, b2_ref, out_ref,
                h0_ref, g_ref, acache_ref, *, bm, nm, ck, f):
    i = pl.program_id(0)

    @pl.when(i == 0)
    def _():
        h0_ref[...] = jnp.dot(
            x_ref[...], w1_ref[...],
            preferred_element_type=jnp.float32).astype(jnp.bfloat16)

    @pl.when(i < nm)
    def _():
        m = i
        s = jnp.dot(a_ref[...].astype(jnp.bfloat16), h0_ref[...],
                    preferred_element_type=jnp.float32)
        s = jnp.maximum(s + b1_ref[...], 0.0)
        g_ref[pl.ds(m * bm, bm), :] = jnp.dot(
            s, w2_ref[...],
            preferred_element_type=jnp.float32).astype(jnp.bfloat16)

        @pl.when(jnp.logical_and(m >= nm - 1 - ck, m <= nm - 2))
        def _():
            base = (m - (nm - 1 - ck)) * bm
            nn = nm * bm
            for c in range(0, nn, 2000):
                hi = min(c + 2000, nn)
                acache_ref[pl.ds(base, bm), c:hi] = (
                    a_ref[:, c:hi].astype(jnp.bfloat16))

    @pl.when(i >= nm)
    def _():
        j = i - nm
        gb = g_ref[...]

        @pl.when(jnp.logical_or(j == 0, j > ck))
        def _():
            s = jnp.dot(a_ref[...].astype(jnp.bfloat16), gb,
                        preferred_element_type=jnp.float32)
            out_ref[...] = jnp.maximum(s + b2_ref[...], 0.0)

        @pl.when(jnp.logical_and(j >= 1, j <= ck))
        def _():
            ab = acache_ref[pl.ds((ck - j) * bm, bm), :]
            s = jnp.dot(ab, gb, preferred_element_type=jnp.float32)
            out_ref[...] = jnp.maximum(s + b2_ref[...], 0.0)


def _phase1_m(j, nm, ck):
    return jnp.where(j <= ck, nm - 1 - j, j - ck - 1)


def kernel(x, a, W1, b1, W2, b2):
    n, f_in = x.shape
    f = W1.shape[1]
    bm = 400
    nm = n // bm
    ck = min(2, nm - 2)

    def a_map(i, nm=nm, ck=ck):
        j = i - nm
        return (jnp.where(i < nm, i,
                          jnp.where(j <= ck, nm - 1, j - ck - 1)), 0)

    def out_map(i, nm=nm, ck=ck):
        return (jnp.where(i < nm, nm - 1, _phase1_m(i - nm, nm, ck)), 0)

    return pl.pallas_call(
        functools.partial(_gcn_kernel, bm=bm, nm=nm, ck=ck, f=f),
        grid=(2 * nm,),
        in_specs=[
            pl.BlockSpec((n, f_in), lambda i: (0, 0),
                         pipeline_mode=pl.Buffered(buffer_count=1)),
            pl.BlockSpec((bm, n), a_map,
                         pipeline_mode=pl.Buffered(buffer_count=2)),
            pl.BlockSpec((f_in, f), lambda i: (0, 0),
                         pipeline_mode=pl.Buffered(buffer_count=1)),
            pl.BlockSpec((1, f), lambda i: (0, 0),
                         pipeline_mode=pl.Buffered(buffer_count=1)),
            pl.BlockSpec((f, f), lambda i: (0, 0),
                         pipeline_mode=pl.Buffered(buffer_count=1)),
            pl.BlockSpec((1, f), lambda i: (0, 0),
                         pipeline_mode=pl.Buffered(buffer_count=1)),
        ],
        out_specs=pl.BlockSpec((bm, f), out_map),
        out_shape=jax.ShapeDtypeStruct((n, f), jnp.float32),
        scratch_shapes=[
            pltpu.VMEM((n, f), jnp.bfloat16),
            pltpu.VMEM((n, f), jnp.bfloat16),
            pltpu.VMEM((ck * bm, n), jnp.bfloat16),
        ],
        compiler_params=pltpu.CompilerParams(
            dimension_semantics=("arbitrary",),
            vmem_limit_bytes=64 * 1024 * 1024,
            internal_scratch_in_bytes=256 * 1024,
        ),
    )(x, a, W1, b1.reshape(1, f), W2, b2.reshape(1, f))
